# baseline (speedup 1.0000x reference)
import functools

import jax
import jax.numpy as jnp
from jax import lax
from jax.experimental import pallas as pl
from jax.experimental.pallas import tpu as pltpu, tpu_sc as plsc

_GAMMA = 0.1

_ROWS, _LANES = 12500, 16
_NW = 32
_BIG = 392
_NBIG = 31
_SMALL = 348


def _scale_chunk(v_hbm, o_hbm, buf, base, rows):
    pltpu.sync_copy(v_hbm.at[pl.ds(base, rows)], buf.at[pl.ds(0, rows)])

    def body(i, _):
        buf[i] = buf[i] * (-_GAMMA)
        return 0

    lax.fori_loop(0, rows, body, 0)
    pltpu.sync_copy(buf.at[pl.ds(0, rows)], o_hbm.at[pl.ds(base, rows)])


@functools.partial(
    pl.kernel,
    mesh=plsc.VectorSubcoreMesh(core_axis_name="c", subcore_axis_name="s"),
    out_type=jax.ShapeDtypeStruct((_ROWS, _LANES), jnp.float32),
    scratch_types=[pltpu.VMEM((_BIG, _LANES), jnp.float32)],
    compiler_params=pltpu.CompilerParams(use_tc_tiling_on_sc=False),
)
def _damp_sc(v_hbm, o_hbm, buf):
    wid = lax.axis_index("s") * 2 + lax.axis_index("c")
    base = pl.multiple_of(wid * _BIG, 8)

    @pl.when(wid < _NBIG)
    def _():
        _scale_chunk(v_hbm, o_hbm, buf, base, _BIG)

    @pl.when(wid >= _NBIG)
    def _():
        _scale_chunk(v_hbm, o_hbm, buf, base, _SMALL)


def kernel(x, v, edge_index):
    n = v.shape[0]
    o = _damp_sc(v.reshape(_ROWS, _LANES))
    return o.reshape(n, 2)


# speedup vs baseline: 1.9560x; 1.9560x over previous
import jax
import jax.numpy as jnp
from jax.experimental import pallas as pl

_GAMMA = 0.1
_BLOCK = 25000


def _damp_kernel(v_ref, o_ref):
    o_ref[...] = v_ref[...] * (-_GAMMA)


def kernel(x, v, edge_index):
    n = v.shape[0]
    return pl.pallas_call(
        _damp_kernel,
        grid=(n // _BLOCK,),
        in_specs=[pl.BlockSpec((_BLOCK, 2), lambda i: (i, 0))],
        out_specs=pl.BlockSpec((_BLOCK, 2), lambda i: (i, 0)),
        out_shape=jax.ShapeDtypeStruct(v.shape, v.dtype),
    )(v)
